# baseline (speedup 1.0000x reference)
import functools

import jax
import jax.numpy as jnp
from jax import lax
from jax.experimental import pallas as pl
from jax.experimental.pallas import tpu as pltpu
from jax.experimental.pallas import tpu_sc as plsc

B = 128
N = 32768
K = 256
L = 16
ABS_MASK = 0x7FFFFFFF
SENTINEL = 0x7FFFFFFF


def _zero_hist(hist_v, nbins):
    zeros = jnp.zeros((L,), jnp.int32)

    @plsc.parallel_loop(0, nbins, unroll=8)
    def _(i):
        hist_v[pl.ds(i * L, L)] = zeros


def _scan_hist(hist_v, nbins, remaining):
    ngroups = nbins // L

    def group_body(j, carry):
        rem, found, gsel = carry
        g = ngroups - 1 - j
        acc = hist_v[pl.ds(g * L * L, L)]
        for t in range(1, L):
            acc = acc + hist_v[pl.ds(g * L * L + t * L, L)]
        s = jnp.sum(acc)
        take = jnp.logical_and(found == 0, s >= rem)
        gsel = jnp.where(take, g, gsel)
        rem = jnp.where(jnp.logical_or(found == 1, take), rem, rem - s)
        found = jnp.where(take, jnp.int32(1), found)
        return rem, found, gsel

    remaining, _, gsel = lax.fori_loop(
        0, ngroups, group_body, (remaining, jnp.int32(0), jnp.int32(0)),
        unroll=4)

    def bin_body(j, carry):
        rem, found, bsel, ssel = carry
        b = gsel * L + (L - 1 - j)
        s = jnp.sum(hist_v[pl.ds(b * L, L)])
        take = jnp.logical_and(found == 0, s >= rem)
        bsel = jnp.where(take, b, bsel)
        ssel = jnp.where(take, s, ssel)
        rem = jnp.where(jnp.logical_or(found == 1, take), rem, rem - s)
        found = jnp.where(take, jnp.int32(1), found)
        return rem, found, bsel, ssel

    remaining, _, bsel, ssel = lax.fori_loop(
        0, L, bin_body,
        (remaining, jnp.int32(0), jnp.int32(0), jnp.int32(0)), unroll=4)
    return bsel, remaining, ssel


def _row_threshold(row_v, cbuf_v, hist_v, lane):
    ones = jnp.ones((L,), jnp.int32)

    _zero_hist(hist_v, 256)

    @plsc.parallel_loop(0, N // L, unroll=16)
    def _(i):
        x = row_v[pl.ds(i * L, L)]
        u = lax.bitcast_convert_type(x, jnp.int32) & ABS_MASK
        idx = (lax.shift_right_logical(u, 19) & 0xFF0) | lane
        plsc.addupdate_scatter(hist_v, [idx], ones)

    b1, rem, _ = _scan_hist(hist_v, 256, jnp.int32(K))

    _zero_hist(hist_v, 256)

    @plsc.parallel_loop(0, N // L, unroll=16, carry=jnp.zeros((L,), jnp.int32))
    def off_l(i, off_l):
        x = row_v[pl.ds(i * L, L)]
        u = lax.bitcast_convert_type(x, jnp.int32) & ABS_MASK
        active = lax.shift_right_logical(u, 23) == b1
        idx = (lax.shift_right_logical(u, 11) & 0xFF0) | lane
        plsc.addupdate_scatter(hist_v, [idx], ones, mask=active)
        dest = lax.shift_left(off_l, 4) | lane
        plsc.store_scatter(cbuf_v, [dest], u, mask=active)
        return off_l + jnp.where(active, jnp.int32(1), jnp.int32(0))

    b2, rem, _ = _scan_hist(hist_v, 256, rem)
    pfx2 = (b1 << 8) | b2

    _zero_hist(hist_v, 256)
    iters_c = jnp.max(off_l)

    def l3_body(j, _):
        u = cbuf_v[pl.ds(j * L, L)]
        inb = j < off_l
        active = jnp.logical_and(
            inb, lax.shift_right_logical(u, 15) == pfx2)
        idx = (lax.shift_right_logical(u, 3) & 0xFF0) | lane
        plsc.addupdate_scatter(hist_v, [idx], ones, mask=active)
        return 0

    lax.fori_loop(0, iters_c, l3_body, 0)
    b3, rem, _ = _scan_hist(hist_v, 256, rem)
    pfx3 = (pfx2 << 8) | b3

    _zero_hist(hist_v, 128)

    def l4_body(j, _):
        u = cbuf_v[pl.ds(j * L, L)]
        inb = j < off_l
        active = jnp.logical_and(
            inb, lax.shift_right_logical(u, 7) == pfx3)
        idx = ((u & 0x7F) << 4) | lane
        plsc.addupdate_scatter(hist_v, [idx], ones, mask=active)
        return 0

    lax.fori_loop(0, iters_c, l4_body, 0)
    b4, rem4, s4 = _scan_hist(hist_v, 128, rem)
    return (pfx3 << 7) | b4, rem4, s4


def _tec_body(in_hbm, out_hbm, rowa_v, rowb_v, cbuf_v, hist_v,
              sia, sib, soa, sob):
    nc = 2
    rows_per_w = B // (nc * 16)
    wid = lax.axis_index("s") * nc + lax.axis_index("c")
    base = wid * rows_per_w
    lane = lax.iota(jnp.int32, 16)
    zeros = jnp.zeros((L,), jnp.float32)

    bufs = (rowa_v, rowb_v)
    sin = (sia, sib)
    sout = (soa, sob)

    pltpu.async_copy(in_hbm.at[base], bufs[0], sin[0])
    out_pending = [False, False]

    for rr in range(rows_per_w):
        p = rr % 2
        cur = bufs[p]
        pltpu.make_async_copy(in_hbm.at[base + rr], cur, sin[p]).wait()
        if rr + 1 < rows_per_w:
            q = 1 - p
            if out_pending[q]:
                pltpu.make_async_copy(
                    bufs[q], out_hbm.at[base + rr - 1], sout[q]).wait()
                out_pending[q] = False
            pltpu.async_copy(in_hbm.at[base + rr + 1], bufs[q], sin[q])

        thresh, n_keep_eq, n_eq = _row_threshold(cur, cbuf_v, hist_v, lane)

        @pl.when(n_keep_eq == n_eq)
        def _():
            @plsc.parallel_loop(0, N // L, unroll=16)
            def _(i):
                x = cur[pl.ds(i * L, L)]
                u = lax.bitcast_convert_type(x, jnp.int32) & ABS_MASK
                cur[pl.ds(i * L, L)] = jnp.where(u >= thresh, x, zeros)

        @pl.when(n_keep_eq != n_eq)
        def _():
            def tie_body(i, seen):
                x = cur[pl.ds(i * L, L)]
                u = lax.bitcast_convert_type(x, jnp.int32) & ABS_MASK
                eq = u == thresh
                pfs = plsc.cumsum(jnp.where(eq, jnp.int32(1), jnp.int32(0)))
                keep = jnp.logical_or(
                    u > thresh,
                    jnp.logical_and(eq, seen + pfs <= n_keep_eq))
                cur[pl.ds(i * L, L)] = jnp.where(keep, x, zeros)
                return seen + plsc.all_reduce_population_count(eq)

            lax.fori_loop(
                0, N // L, tie_body, jnp.zeros((L,), jnp.int32), unroll=4)

        pltpu.async_copy(cur, out_hbm.at[base + rr], sout[p])
        out_pending[p] = True

    for rr in (rows_per_w - 2, rows_per_w - 1):
        p = rr % 2
        if out_pending[p]:
            pltpu.make_async_copy(
                bufs[p], out_hbm.at[base + rr], sout[p]).wait()
            out_pending[p] = False


@jax.jit
def kernel(input_):
    mesh = plsc.VectorSubcoreMesh(core_axis_name="c", subcore_axis_name="s")
    f = functools.partial(
        pl.kernel,
        mesh=mesh,
        out_type=jax.ShapeDtypeStruct((B, N), jnp.float32),
        scratch_types=[
            pltpu.VMEM((N,), jnp.float32),
            pltpu.VMEM((N,), jnp.float32),
            pltpu.VMEM((N + L,), jnp.int32),
            pltpu.VMEM((256 * L,), jnp.int32),
            pltpu.SemaphoreType.DMA,
            pltpu.SemaphoreType.DMA,
            pltpu.SemaphoreType.DMA,
            pltpu.SemaphoreType.DMA,
        ],
        compiler_params=pltpu.CompilerParams(needs_layout_passes=False),
    )(_tec_body)
    return f(input_)


# speedup vs baseline: 1.0204x; 1.0204x over previous
import functools

import jax
import jax.numpy as jnp
from jax import lax
from jax.experimental import pallas as pl
from jax.experimental.pallas import tpu as pltpu
from jax.experimental.pallas import tpu_sc as plsc

B = 128
N = 32768
K = 256
L = 16
ABS_MASK = 0x7FFFFFFF
SENTINEL = 0x7FFFFFFF


def _zero_hist(hist_v, nbins):
    zeros = jnp.zeros((L,), jnp.int32)

    @plsc.parallel_loop(0, nbins, unroll=8)
    def _(i):
        hist_v[pl.ds(i * L, L)] = zeros


def _scan_hist(hist_v, nbins, remaining):
    ngroups = nbins // L

    def group_body(j, carry):
        rem, found, gsel = carry
        g = ngroups - 1 - j
        acc = hist_v[pl.ds(g * L * L, L)]
        for t in range(1, L):
            acc = acc + hist_v[pl.ds(g * L * L + t * L, L)]
        s = jnp.sum(acc)
        take = jnp.logical_and(found == 0, s >= rem)
        gsel = jnp.where(take, g, gsel)
        rem = jnp.where(jnp.logical_or(found == 1, take), rem, rem - s)
        found = jnp.where(take, jnp.int32(1), found)
        return rem, found, gsel

    remaining, _, gsel = lax.fori_loop(
        0, ngroups, group_body, (remaining, jnp.int32(0), jnp.int32(0)),
        unroll=4)

    def bin_body(j, carry):
        rem, found, bsel, ssel = carry
        b = gsel * L + (L - 1 - j)
        s = jnp.sum(hist_v[pl.ds(b * L, L)])
        take = jnp.logical_and(found == 0, s >= rem)
        bsel = jnp.where(take, b, bsel)
        ssel = jnp.where(take, s, ssel)
        rem = jnp.where(jnp.logical_or(found == 1, take), rem, rem - s)
        found = jnp.where(take, jnp.int32(1), found)
        return rem, found, bsel, ssel

    remaining, _, bsel, ssel = lax.fori_loop(
        0, L, bin_body,
        (remaining, jnp.int32(0), jnp.int32(0), jnp.int32(0)), unroll=4)
    return bsel, remaining, ssel


def _row_threshold(row_v, cbuf_v, hist_v, lane):
    ones = jnp.ones((L,), jnp.int32)

    _zero_hist(hist_v, 256)

    @plsc.parallel_loop(0, N // L, unroll=16)
    def _(i):
        x = row_v[pl.ds(i * L, L)]
        u = lax.bitcast_convert_type(x, jnp.int32) & ABS_MASK
        idx = (lax.shift_right_logical(u, 19) & 0xFF0) | lane
        plsc.addupdate_scatter(hist_v, [idx], ones)

    b1, rem, _ = _scan_hist(hist_v, 256, jnp.int32(K))

    _zero_hist(hist_v, 256)

    @plsc.parallel_loop(0, N // L, unroll=16, carry=jnp.zeros((L,), jnp.int32))
    def off_l(i, off_l):
        x = row_v[pl.ds(i * L, L)]
        u = lax.bitcast_convert_type(x, jnp.int32) & ABS_MASK
        active = lax.shift_right_logical(u, 23) == b1
        dest = lax.shift_left(off_l, 4) | lane
        plsc.store_scatter(cbuf_v, [dest], u, mask=active)
        return off_l + jnp.where(active, jnp.int32(1), jnp.int32(0))

    iters_c = jnp.max(off_l)

    def l2_body(j, _):
        u = cbuf_v[pl.ds(j * L, L)]
        inb = j < off_l
        idx = (lax.shift_right_logical(u, 11) & 0xFF0) | lane
        plsc.addupdate_scatter(hist_v, [idx], ones, mask=inb)
        return 0

    lax.fori_loop(0, iters_c, l2_body, 0)

    b2, rem, _ = _scan_hist(hist_v, 256, rem)
    pfx2 = (b1 << 8) | b2

    _zero_hist(hist_v, 256)

    def l3_body(j, _):
        u = cbuf_v[pl.ds(j * L, L)]
        inb = j < off_l
        active = jnp.logical_and(
            inb, lax.shift_right_logical(u, 15) == pfx2)
        idx = (lax.shift_right_logical(u, 3) & 0xFF0) | lane
        plsc.addupdate_scatter(hist_v, [idx], ones, mask=active)
        return 0

    lax.fori_loop(0, iters_c, l3_body, 0)
    b3, rem, _ = _scan_hist(hist_v, 256, rem)
    pfx3 = (pfx2 << 8) | b3

    _zero_hist(hist_v, 128)

    def l4_body(j, _):
        u = cbuf_v[pl.ds(j * L, L)]
        inb = j < off_l
        active = jnp.logical_and(
            inb, lax.shift_right_logical(u, 7) == pfx3)
        idx = ((u & 0x7F) << 4) | lane
        plsc.addupdate_scatter(hist_v, [idx], ones, mask=active)
        return 0

    lax.fori_loop(0, iters_c, l4_body, 0)
    b4, rem4, s4 = _scan_hist(hist_v, 128, rem)
    return (pfx3 << 7) | b4, rem4, s4


def _tec_body(in_hbm, out_hbm, rowa_v, rowb_v, cbuf_v, hist_v,
              sia, sib, soa, sob):
    nc = 2
    rows_per_w = B // (nc * 16)
    wid = lax.axis_index("s") * nc + lax.axis_index("c")
    base = wid * rows_per_w
    lane = lax.iota(jnp.int32, 16)
    zeros = jnp.zeros((L,), jnp.float32)

    bufs = (rowa_v, rowb_v)
    sin = (sia, sib)
    sout = (soa, sob)

    pltpu.async_copy(in_hbm.at[base], bufs[0], sin[0])
    out_pending = [False, False]

    for rr in range(rows_per_w):
        p = rr % 2
        cur = bufs[p]
        pltpu.make_async_copy(in_hbm.at[base + rr], cur, sin[p]).wait()
        if rr + 1 < rows_per_w:
            q = 1 - p
            if out_pending[q]:
                pltpu.make_async_copy(
                    bufs[q], out_hbm.at[base + rr - 1], sout[q]).wait()
                out_pending[q] = False
            pltpu.async_copy(in_hbm.at[base + rr + 1], bufs[q], sin[q])

        thresh, n_keep_eq, n_eq = _row_threshold(cur, cbuf_v, hist_v, lane)

        @pl.when(n_keep_eq == n_eq)
        def _():
            @plsc.parallel_loop(0, N // L, unroll=16)
            def _(i):
                x = cur[pl.ds(i * L, L)]
                u = lax.bitcast_convert_type(x, jnp.int32) & ABS_MASK
                cur[pl.ds(i * L, L)] = jnp.where(u >= thresh, x, zeros)

        @pl.when(n_keep_eq != n_eq)
        def _():
            def tie_body(i, seen):
                x = cur[pl.ds(i * L, L)]
                u = lax.bitcast_convert_type(x, jnp.int32) & ABS_MASK
                eq = u == thresh
                pfs = plsc.cumsum(jnp.where(eq, jnp.int32(1), jnp.int32(0)))
                keep = jnp.logical_or(
                    u > thresh,
                    jnp.logical_and(eq, seen + pfs <= n_keep_eq))
                cur[pl.ds(i * L, L)] = jnp.where(keep, x, zeros)
                return seen + plsc.all_reduce_population_count(eq)

            lax.fori_loop(
                0, N // L, tie_body, jnp.zeros((L,), jnp.int32), unroll=4)

        pltpu.async_copy(cur, out_hbm.at[base + rr], sout[p])
        out_pending[p] = True

    for rr in (rows_per_w - 2, rows_per_w - 1):
        p = rr % 2
        if out_pending[p]:
            pltpu.make_async_copy(
                bufs[p], out_hbm.at[base + rr], sout[p]).wait()
            out_pending[p] = False


@jax.jit
def kernel(input_):
    mesh = plsc.VectorSubcoreMesh(core_axis_name="c", subcore_axis_name="s")
    f = functools.partial(
        pl.kernel,
        mesh=mesh,
        out_type=jax.ShapeDtypeStruct((B, N), jnp.float32),
        scratch_types=[
            pltpu.VMEM((N,), jnp.float32),
            pltpu.VMEM((N,), jnp.float32),
            pltpu.VMEM((N + L,), jnp.int32),
            pltpu.VMEM((256 * L,), jnp.int32),
            pltpu.SemaphoreType.DMA,
            pltpu.SemaphoreType.DMA,
            pltpu.SemaphoreType.DMA,
            pltpu.SemaphoreType.DMA,
        ],
        compiler_params=pltpu.CompilerParams(needs_layout_passes=False),
    )(_tec_body)
    return f(input_)
